# Initial kernel scaffold; baseline (speedup 1.0000x reference)
import jax
import jax.numpy as jnp
from jax.experimental import pallas as pl


def kernel(X, g, W_self, W_neigh, b):
    raise NotImplementedError("write your pallas kernel here")


# speedup vs baseline: 24.3438x; 24.3438x over previous
import functools

import jax
import jax.numpy as jnp
from jax import lax
from jax.experimental import pallas as pl
from jax.experimental.pallas import tpu as pltpu
from jax.experimental.pallas import tpu_sc as plsc

N = 10000
E = 160000
F = 128
FP = 144
NREP = 4
KE = 128
NCHUNKS = E // KE
ROWS = 624


def _sc_aggregate(xaug, src, dst, zrows):
  mesh = plsc.VectorSubcoreMesh(core_axis_name="c", subcore_axis_name="s")

  @functools.partial(
      pl.kernel,
      mesh=mesh,
      compiler_params=pltpu.CompilerParams(use_tc_tiling_on_sc=False),
      out_type=jax.ShapeDtypeStruct((NREP * N, FP), jnp.float32),
      scratch_types=[
          pltpu.VMEM((KE,), jnp.int32),
          pltpu.VMEM((KE,), jnp.int32),
          pltpu.VMEM((KE,), jnp.int32),
          pltpu.VMEM((KE, FP), jnp.float32),
          pltpu.VMEM_SHARED((N, FP), jnp.float32),
          pltpu.SemaphoreType.DMA,
      ],
  )
  def k(xaug_hbm, src_hbm, dst_hbm, z_hbm, out_hbm,
        si_v, so_v, di_v, rows_v, acc, sem):
    c = lax.axis_index("c")
    s = lax.axis_index("s")
    row_base = s * ROWS
    nch = jnp.where(s < NCHUNKS - (NCHUNKS // 16) * 16, NCHUNKS // 16 + 1,
                    NCHUNKS // 16)

    for rr in range(NREP // 2):
      r = rr * 2 + c
      roff = r * N

      pltpu.sync_copy(z_hbm.at[pl.ds(0, ROWS)],
                      acc.at[pl.ds(row_base, ROWS)])

      @pl.when(s == 15)
      def _():
        pltpu.sync_copy(z_hbm.at[pl.ds(0, 16)], acc.at[pl.ds(15 * ROWS + ROWS, 16)])

      plsc.subcore_barrier()

      def chunk(i, carry):
        e0 = (i * 16 + s) * KE
        pltpu.sync_copy(src_hbm.at[pl.ds(e0, KE)], si_v)
        pltpu.sync_copy(dst_hbm.at[pl.ds(e0, KE)], di_v)
        for j in range(KE // 16):
          sl = pl.ds(j * 16, 16)
          so_v[sl] = si_v[sl] + roff
        pltpu.async_copy(xaug_hbm.at[so_v], rows_v, sem).wait()
        pltpu.sync_copy(rows_v, acc.at[di_v], add=True)
        return carry

      lax.fori_loop(0, nch, chunk, 0)

      plsc.subcore_barrier()

      pltpu.sync_copy(acc.at[pl.ds(row_base, ROWS)],
                      out_hbm.at[pl.ds(roff + row_base, ROWS)])

      @pl.when(s == 15)
      def _():
        pltpu.sync_copy(acc.at[pl.ds(16 * ROWS, 16)],
                        out_hbm.at[pl.ds(roff + 16 * ROWS, 16)])

      plsc.subcore_barrier()

  return k(xaug, src, dst, zrows)


def _tc_body(x_ref, agg_ref, wn_ref, ws_ref, b_ref, o_ref):
  wn = wn_ref[...]
  ws = ws_ref[...]
  bias = b_ref[0, :]
  for t in range(2):
    x = x_ref[0, :, t, :]
    ag = agg_ref[t, :, 0:F]
    deg = agg_ref[t, :, F:F + 1]
    mean = ag / jnp.maximum(deg, 1.0)
    acc = jnp.dot(mean, wn, preferred_element_type=jnp.float32,
                  precision=lax.Precision.HIGHEST)
    acc += jnp.dot(x, ws, preferred_element_type=jnp.float32,
                   precision=lax.Precision.HIGHEST)
    o_ref[0, :, t, :] = jnp.maximum(acc + bias, 0.0)


def kernel(X, g, W_self, W_neigh, b):
  B, n, T, f = X.shape
  src = g[0]
  dst = g[1]

  t1 = jnp.transpose(X, (0, 2, 1, 3)).reshape(NREP * N, F)
  xaug = jnp.concatenate(
      [t1,
       jnp.ones((NREP * N, 1), jnp.float32),
       jnp.zeros((NREP * N, FP - F - 1), jnp.float32)], axis=1)
  zrows = jnp.zeros((ROWS, FP), jnp.float32)

  agg = _sc_aggregate(xaug, src, dst, zrows).reshape(NREP, N, FP)

  RB = 2000
  out = pl.pallas_call(
      _tc_body,
      grid=(B, N // RB),
      in_specs=[
          pl.BlockSpec((1, RB, T, F), lambda bb, nn: (bb, nn, 0, 0)),
          pl.BlockSpec((T, RB, FP), lambda bb, nn: (bb, nn, 0)),
          pl.BlockSpec((F, F), lambda bb, nn: (0, 0)),
          pl.BlockSpec((F, F), lambda bb, nn: (0, 0)),
          pl.BlockSpec((1, F), lambda bb, nn: (0, 0)),
      ],
      out_specs=pl.BlockSpec((1, RB, T, F), lambda bb, nn: (bb, nn, 0, 0)),
      out_shape=jax.ShapeDtypeStruct((B, N, T, F), jnp.float32),
  )(X, agg, W_neigh.T, W_self.T, b.reshape(1, F))
  return out
